# baseline (speedup 1.0000x reference)
import jax
import jax.numpy as jnp
import numpy as np
from jax import lax
from jax.experimental import pallas as pl
from jax.experimental.pallas import tpu as pltpu
from jax.experimental.pallas import tpu_sc as plsc

N_BINS = 15
N = 4194304
N_SC = 2097152
N_TC = N - N_SC
L = 16
NC = 2
NS = 16
NW = NC * NS
W = N_SC // NW
CHUNK = 16384
NBUF = 2
N_CHUNKS = W // CHUNK
TC_BLK = 65536
TC_R = TC_BLK // 128
TC_STEPS = N_TC // TC_BLK
TC_ROW0 = N_SC // 128 // TC_R
BOUNDS = [float(x) for x in np.linspace(0.0, 1.0, 16, dtype=np.float32)]
VECS = CHUNK // L
HIST = L * N_BINS
NTAB = 8
UN = 16
PACK_SHIFT = 17


def _sc_body(conf_hbm, pred_hbm, lab_hbm, out_i_hbm, out_c_hbm,
             conf_v, pred_v, lab_v, hist_i, hist_c, sems):
  wid = lax.axis_index("s") * NC + lax.axis_index("c")
  base = wid * W

  lane = jnp.arange(L, dtype=jnp.int32)
  lane_k = [lane + jnp.int32(k * HIST) for k in range(NTAB)]
  zero_i = jnp.zeros((L,), jnp.int32)
  zero_f = jnp.zeros((L,), jnp.float32)
  for b in range(NTAB * N_BINS):
    hist_i[pl.ds(b * L, L)] = zero_i
    hist_c[pl.ds(b * L, L)] = zero_f

  def copies(slot, ci):
    off = base + ci * CHUNK
    sl = pl.ds(off, CHUNK)
    return (
        pltpu.make_async_copy(conf_hbm.at[sl], conf_v.at[slot], sems.at[slot]),
        pltpu.make_async_copy(pred_hbm.at[sl], pred_v.at[slot], sems.at[slot]),
        pltpu.make_async_copy(lab_hbm.at[sl], lab_v.at[slot], sems.at[slot]),
    )

  for slot in range(NBUF):
    for cp in copies(slot, slot):
      cp.start()

  zero_f16 = jnp.zeros((L,), jnp.float32)
  max_bin = jnp.full((L,), N_BINS - 1, jnp.int32)
  pack0 = jnp.full((L,), 1 << PACK_SHIFT, jnp.int32)
  pack1 = jnp.full((L,), (1 << PACK_SHIFT) + 1, jnp.int32)
  nbins_f = jnp.full((L,), float(N_BINS), jnp.float32)

  def process(slot):
    @plsc.parallel_loop(0, VECS // UN, unroll=1)
    def vec_group(g):
      s0 = g * (L * UN)
      cs, ps, ys = [], [], []
      for k in range(UN):
        s = s0 + k * L
        cs.append(conf_v[slot, pl.ds(s, L)])
        ps.append(pred_v[slot, pl.ds(s, L)])
        ys.append(lab_v[slot, pl.ds(s, L)])
      addrs, packs, valids = [], [], []
      for k in range(UN):
        ti = (cs[k] * nbins_f).astype(jnp.int32)
        bin_ = jnp.minimum(ti, max_bin)
        valids.append(cs[k] > zero_f16)
        addrs.append(lane_k[k % NTAB] + (bin_ << 4))
        packs.append(jnp.where(ps[k] == ys[k], pack1, pack0))
      for k in range(UN):
        plsc.addupdate_scatter(hist_i, [addrs[k]], packs[k], mask=valids[k])
        plsc.addupdate_scatter(hist_c, [addrs[k]], cs[k], mask=valids[k])

  def round_body(k, _):
    ci0 = k * NBUF
    for slot in range(NBUF):
      ci = ci0 + slot
      for cp in copies(slot, ci):
        cp.wait()
      nxt = ci + NBUF

      @pl.when(nxt < N_CHUNKS)
      def _():
        for cp in copies(slot, nxt):
          cp.start()

      process(slot)
    return ()

  lax.fori_loop(0, N_CHUNKS // NBUF, round_body, ())

  for pos in range(N_BINS):
    ai = hist_i[pl.ds(pos * L, L)]
    ac = hist_c[pl.ds(pos * L, L)]
    for k in range(1, NTAB):
      ai = ai + hist_i[pl.ds(k * HIST + pos * L, L)]
      ac = ac + hist_c[pl.ds(k * HIST + pos * L, L)]
    hist_i[pl.ds(pos * L, L)] = ai
    hist_c[pl.ds(pos * L, L)] = ac

  pltpu.sync_copy(hist_i.at[pl.ds(0, HIST)],
                  out_i_hbm.at[pl.ds(wid * HIST, HIST)])
  pltpu.sync_copy(hist_c.at[pl.ds(0, HIST)],
                  out_c_hbm.at[pl.ds(wid * HIST, HIST)])


def _tc_hist_body(c_ref, p_ref, y_ref, out_ref):
  i = pl.program_id(0)
  c = c_ref[...]
  acc = (p_ref[...] == y_ref[...]).astype(jnp.float32)
  zero = jnp.zeros_like(c)
  xn, xs, xa = [], [], []
  for b in BOUNDS:
    m = c > b
    xn.append(jnp.sum(jnp.where(m, 1.0, 0.0)))
    xs.append(jnp.sum(jnp.where(m, c, zero)))
    xa.append(jnp.sum(jnp.where(m, acc, zero)))
  rows = jnp.stack([jnp.stack(xn), jnp.stack(xs), jnp.stack(xa)])

  @pl.when(i == 0)
  def _():
    out_ref[...] = jnp.zeros(out_ref.shape, out_ref.dtype)

  out_ref[...] += rows


def _combine_body(pi_ref, pc_ref, tc_ref, out_ref):
  vi = pi_ref[...]
  counts = (vi >> PACK_SHIFT).astype(jnp.float32)
  accs = (vi & jnp.int32((1 << PACK_SHIFT) - 1)).astype(jnp.float32)
  confs = pc_ref[...]
  cw = jnp.sum(counts, axis=0)
  aw = jnp.sum(accs, axis=0)
  sw = jnp.sum(confs, axis=0)
  tc = tc_ref[...]
  tc_cnt = tc[0, :N_BINS] - tc[0, 1:N_BINS + 1]
  tc_csum = tc[1, :N_BINS] - tc[1, 1:N_BINS + 1]
  tc_acc = tc[2, :N_BINS] - tc[2, 1:N_BINS + 1]
  cnt = jnp.stack([jnp.sum(cw[b * L:(b + 1) * L]) for b in range(N_BINS)]) + tc_cnt
  acc = jnp.stack([jnp.sum(aw[b * L:(b + 1) * L]) for b in range(N_BINS)]) + tc_acc
  csum = jnp.stack([jnp.sum(sw[b * L:(b + 1) * L]) for b in range(N_BINS)]) + tc_csum
  safe = jnp.maximum(cnt, 1.0)
  prop = cnt * jnp.float32(1.0 / N)
  contrib = jnp.abs(csum / safe - acc / safe) * prop
  contrib = jnp.where(prop > 0.0, contrib, 0.0)
  out_ref[0] = jnp.sum(contrib)


@jax.jit
def kernel(confidences, predictions, labels):
  mesh = plsc.VectorSubcoreMesh(core_axis_name="c", subcore_axis_name="s")
  sc = pl.kernel(
      _sc_body,
      out_type=(
          jax.ShapeDtypeStruct((NW * HIST,), jnp.int32),
          jax.ShapeDtypeStruct((NW * HIST,), jnp.float32),
      ),
      mesh=mesh,
      compiler_params=pltpu.CompilerParams(needs_layout_passes=False),
      scratch_types=[
          pltpu.VMEM((NBUF, CHUNK), jnp.float32),
          pltpu.VMEM((NBUF, CHUNK), jnp.int32),
          pltpu.VMEM((NBUF, CHUNK), jnp.int32),
          pltpu.VMEM((NTAB * HIST,), jnp.int32),
          pltpu.VMEM((NTAB * HIST,), jnp.float32),
          pltpu.SemaphoreType.DMA((NBUF,)),
      ],
  )
  part_i, part_c = sc(confidences, predictions, labels)
  part_i = part_i.reshape(NW, HIST)
  part_c = part_c.reshape(NW, HIST)
  c2 = confidences.reshape(N // 128, 128)
  p2 = predictions.reshape(N // 128, 128)
  y2 = labels.reshape(N // 128, 128)
  blk = lambda i: (TC_ROW0 + i, 0)
  tc_sums = pl.pallas_call(
      _tc_hist_body,
      grid=(TC_STEPS,),
      in_specs=[
          pl.BlockSpec((TC_R, 128), blk),
          pl.BlockSpec((TC_R, 128), blk),
          pl.BlockSpec((TC_R, 128), blk),
      ],
      out_specs=pl.BlockSpec((3, 16), lambda i: (0, 0)),
      out_shape=jax.ShapeDtypeStruct((3, 16), jnp.float32),
  )(c2, p2, y2)
  ece = pl.pallas_call(
      _combine_body,
      out_shape=jax.ShapeDtypeStruct((1,), jnp.float32),
      out_specs=pl.BlockSpec(memory_space=pltpu.SMEM),
  )(part_i, part_c, tc_sums)
  return ece


# speedup vs baseline: 1.3244x; 1.3244x over previous
import jax
import jax.numpy as jnp
import numpy as np
from jax import lax
from jax.experimental import pallas as pl
from jax.experimental.pallas import tpu as pltpu
from jax.experimental.pallas import tpu_sc as plsc

N_BINS = 15
N = 4194304
N_SC = 3145728
N_TC = N - N_SC
L = 16
NC = 2
NS = 16
NW = NC * NS
W = N_SC // NW
CHUNK = 16384
NBUF = 2
N_CHUNKS = W // CHUNK
TC_BLK = 65536
TC_R = TC_BLK // 128
TC_STEPS = N_TC // TC_BLK
TC_ROW0 = N_SC // 128 // TC_R
BOUNDS = [float(x) for x in np.linspace(0.0, 1.0, 16, dtype=np.float32)]
VECS = CHUNK // L
HIST = L * N_BINS
NTAB = 8
UN = 16
PACK_SHIFT = 17
HIST_PAD = 256


def _sc_body(conf_hbm, pred_hbm, lab_hbm, out_i_hbm, out_c_hbm,
             conf_v, pred_v, lab_v, hist_i, hist_c, sems):
  wid = lax.axis_index("s") * NC + lax.axis_index("c")
  base = wid * W

  lane = jnp.arange(L, dtype=jnp.int32)
  lane_k = [lane + jnp.int32(k * HIST) for k in range(NTAB)]
  zero_i = jnp.zeros((L,), jnp.int32)
  zero_f = jnp.zeros((L,), jnp.float32)
  for b in range(NTAB * N_BINS):
    hist_i[pl.ds(b * L, L)] = zero_i
    hist_c[pl.ds(b * L, L)] = zero_f

  def copies(slot, ci):
    off = base + ci * CHUNK
    sl = pl.ds(off, CHUNK)
    return (
        pltpu.make_async_copy(conf_hbm.at[sl], conf_v.at[slot], sems.at[slot]),
        pltpu.make_async_copy(pred_hbm.at[sl], pred_v.at[slot], sems.at[slot]),
        pltpu.make_async_copy(lab_hbm.at[sl], lab_v.at[slot], sems.at[slot]),
    )

  for slot in range(NBUF):
    for cp in copies(slot, slot):
      cp.start()

  zero_f16 = jnp.zeros((L,), jnp.float32)
  max_bin = jnp.full((L,), N_BINS - 1, jnp.int32)
  pack0 = jnp.full((L,), 1 << PACK_SHIFT, jnp.int32)
  pack1 = jnp.full((L,), (1 << PACK_SHIFT) + 1, jnp.int32)
  nbins_f = jnp.full((L,), float(N_BINS), jnp.float32)

  def process(slot):
    @plsc.parallel_loop(0, VECS // UN, unroll=1)
    def vec_group(g):
      s0 = g * (L * UN)
      cs, ps, ys = [], [], []
      for k in range(UN):
        s = s0 + k * L
        cs.append(conf_v[slot, pl.ds(s, L)])
        ps.append(pred_v[slot, pl.ds(s, L)])
        ys.append(lab_v[slot, pl.ds(s, L)])
      addrs, packs, valids = [], [], []
      for k in range(UN):
        ti = (cs[k] * nbins_f).astype(jnp.int32)
        bin_ = jnp.minimum(ti, max_bin)
        valids.append(cs[k] > zero_f16)
        addrs.append(lane_k[k % NTAB] + (bin_ << 4))
        packs.append(jnp.where(ps[k] == ys[k], pack1, pack0))
      for k in range(UN):
        plsc.addupdate_scatter(hist_i, [addrs[k]], packs[k], mask=valids[k])
        plsc.addupdate_scatter(hist_c, [addrs[k]], cs[k], mask=valids[k])

  def round_body(k, _):
    ci0 = k * NBUF
    for slot in range(NBUF):
      ci = ci0 + slot
      for cp in copies(slot, ci):
        cp.wait()
      nxt = ci + NBUF

      @pl.when(nxt < N_CHUNKS)
      def _():
        for cp in copies(slot, nxt):
          cp.start()

      process(slot)
    return ()

  lax.fori_loop(0, N_CHUNKS // NBUF, round_body, ())

  for pos in range(N_BINS):
    ai = hist_i[pl.ds(pos * L, L)]
    ac = hist_c[pl.ds(pos * L, L)]
    for k in range(1, NTAB):
      ai = ai + hist_i[pl.ds(k * HIST + pos * L, L)]
      ac = ac + hist_c[pl.ds(k * HIST + pos * L, L)]
    hist_i[pl.ds(pos * L, L)] = ai
    hist_c[pl.ds(pos * L, L)] = ac

  pltpu.sync_copy(hist_i.at[pl.ds(0, HIST_PAD)], out_i_hbm.at[wid])
  pltpu.sync_copy(hist_c.at[pl.ds(0, HIST_PAD)], out_c_hbm.at[wid])


def _tc_hist_body(c_ref, p_ref, y_ref, out_ref):
  i = pl.program_id(0)
  c = c_ref[...]
  acc = (p_ref[...] == y_ref[...]).astype(jnp.float32)
  zero = jnp.zeros_like(c)
  xn, xs, xa = [], [], []
  for b in BOUNDS:
    m = c > b
    xn.append(jnp.sum(jnp.where(m, 1.0, 0.0)))
    xs.append(jnp.sum(jnp.where(m, c, zero)))
    xa.append(jnp.sum(jnp.where(m, acc, zero)))
  rows = jnp.stack([jnp.stack(xn), jnp.stack(xs), jnp.stack(xa)])

  @pl.when(i == 0)
  def _():
    out_ref[...] = jnp.zeros(out_ref.shape, out_ref.dtype)

  out_ref[...] += rows


def _combine_body(pi_ref, pc_ref, tc_ref, out_ref):
  vi = pi_ref[...]
  counts = (vi >> PACK_SHIFT).astype(jnp.float32)
  accs = (vi & jnp.int32((1 << PACK_SHIFT) - 1)).astype(jnp.float32)
  confs = pc_ref[...]
  cw = jnp.sum(counts, axis=0)
  aw = jnp.sum(accs, axis=0)
  sw = jnp.sum(confs, axis=0)
  tc = tc_ref[...]
  tc_cnt = tc[0, :N_BINS] - tc[0, 1:N_BINS + 1]
  tc_csum = tc[1, :N_BINS] - tc[1, 1:N_BINS + 1]
  tc_acc = tc[2, :N_BINS] - tc[2, 1:N_BINS + 1]
  cnt = jnp.stack([jnp.sum(cw[b * L:(b + 1) * L]) for b in range(N_BINS)]) + tc_cnt
  acc = jnp.stack([jnp.sum(aw[b * L:(b + 1) * L]) for b in range(N_BINS)]) + tc_acc
  csum = jnp.stack([jnp.sum(sw[b * L:(b + 1) * L]) for b in range(N_BINS)]) + tc_csum
  safe = jnp.maximum(cnt, 1.0)
  prop = cnt * jnp.float32(1.0 / N)
  contrib = jnp.abs(csum / safe - acc / safe) * prop
  contrib = jnp.where(prop > 0.0, contrib, 0.0)
  out_ref[0] = jnp.sum(contrib)


@jax.jit
def kernel(confidences, predictions, labels):
  mesh = plsc.VectorSubcoreMesh(core_axis_name="c", subcore_axis_name="s")
  sc = pl.kernel(
      _sc_body,
      out_type=(
          jax.ShapeDtypeStruct((NW, HIST_PAD), jnp.int32),
          jax.ShapeDtypeStruct((NW, HIST_PAD), jnp.float32),
      ),
      mesh=mesh,
      compiler_params=pltpu.CompilerParams(needs_layout_passes=False),
      scratch_types=[
          pltpu.VMEM((NBUF, CHUNK), jnp.float32),
          pltpu.VMEM((NBUF, CHUNK), jnp.int32),
          pltpu.VMEM((NBUF, CHUNK), jnp.int32),
          pltpu.VMEM((NTAB * HIST + HIST_PAD - HIST,), jnp.int32),
          pltpu.VMEM((NTAB * HIST + HIST_PAD - HIST,), jnp.float32),
          pltpu.SemaphoreType.DMA((NBUF,)),
      ],
  )
  part_i, part_c = sc(confidences, predictions, labels)
  c2 = confidences.reshape(N // 128, 128)
  p2 = predictions.reshape(N // 128, 128)
  y2 = labels.reshape(N // 128, 128)
  blk = lambda i: (TC_ROW0 + i, 0)
  tc_sums = pl.pallas_call(
      _tc_hist_body,
      grid=(TC_STEPS,),
      in_specs=[
          pl.BlockSpec((TC_R, 128), blk),
          pl.BlockSpec((TC_R, 128), blk),
          pl.BlockSpec((TC_R, 128), blk),
      ],
      out_specs=pl.BlockSpec((3, 16), lambda i: (0, 0)),
      out_shape=jax.ShapeDtypeStruct((3, 16), jnp.float32),
  )(c2, p2, y2)
  ece = pl.pallas_call(
      _combine_body,
      out_shape=jax.ShapeDtypeStruct((1,), jnp.float32),
      out_specs=pl.BlockSpec(memory_space=pltpu.SMEM),
  )(part_i, part_c, tc_sums)
  return ece
